# baseline (speedup 1.0000x reference)
import jax
import jax.numpy as jnp
from jax.experimental import pallas as pl
from jax.experimental.pallas import tpu as pltpu

_SIZE = 65536
_SHIFT = 8192
_ZLEN = _SIZE - _SHIFT
_ROWS = 64
_NG = 8
_GROW = _ROWS // _NG
_GLEN = _GROW * _SIZE


def _body(x_hbm, o_hbm, zbuf, xbuf, isem, osem):
    icp = pltpu.make_async_copy(x_hbm, xbuf, isem)
    icp.start()
    for r in range(_ROWS):
        zbuf[pl.ds(r * _SIZE, _ZLEN)] = jnp.zeros((_ZLEN,), jnp.float32)
    icp.wait()
    cps = []
    for g in range(_NG):
        for r in range(_GROW):
            row = _GROW * g + r
            zbuf[pl.ds(row * _SIZE + _ZLEN, _SHIFT)] = (
                xbuf[pl.ds(row * _SHIFT, _SHIFT)])
        cps.append(pltpu.async_copy(
            zbuf.at[pl.ds(g * _GLEN, _GLEN)],
            o_hbm.at[pl.ds(g * _GLEN, _GLEN)],
            osem, priority=g % 2))
    for c in cps:
        c.wait()


def kernel(x):
    xf = x.reshape(_ROWS * _SHIFT)
    out = pl.pallas_call(
        _body,
        in_specs=[pl.BlockSpec(memory_space=pl.ANY)],
        out_specs=pl.BlockSpec(memory_space=pl.ANY),
        out_shape=jax.ShapeDtypeStruct((_ROWS * _SIZE,), jnp.float32),
        scratch_shapes=[
            pltpu.VMEM((_ROWS * _SIZE,), jnp.float32),
            pltpu.VMEM((_ROWS * _SHIFT,), jnp.float32),
            pltpu.SemaphoreType.DMA,
            pltpu.SemaphoreType.DMA,
        ],
    )(xf)
    return out.reshape(x.shape[:-1] + (_SIZE,))
